# Initial kernel scaffold; baseline (speedup 1.0000x reference)
import jax
import jax.numpy as jnp
from jax.experimental import pallas as pl


def kernel(xyz, new_xyz, features, idx):
    raise NotImplementedError("write your pallas kernel here")


# speedup vs baseline: 34.2438x; 34.2438x over previous
import functools

import jax
import jax.numpy as jnp
from jax import lax
from jax.experimental import pallas as pl
from jax.experimental.pallas import tpu as pltpu, tpu_sc as plsc

B, N, M, C, K = 8, 65536, 4096, 64, 32
MK = M * K
CH = C + 3
NT = B * CH

NC, NS, L = 2, 16, 16
NW = NC * NS
TASKS_PER_W = (NT + NW - 1) // NW

CHUNK = 16384
NCHUNK = MK // CHUNK
VECS = CHUNK // L
UNROLL = 8


def _qag_kernel(xyz_t, nxyz_t, feat, idx2, nf, gx,
                table_v, idx_v, outa_v, outb_v, cen_v):
    cid = lax.axis_index("c")
    sid = lax.axis_index("s")
    wid = sid * NC + cid

    def gather_chunk(h):
        def body(i, carry):
            base = i * (L * UNROLL)
            for j in range(UNROLL):
                iv = idx_v[pl.ds(base + j * L, L)]
                outa_v[pl.ds(base + j * L, L)] = plsc.load_gather(table_v, [iv])
            return carry
        lax.fori_loop(0, VECS // UNROLL, body, 0, unroll=False)

    def diff_chunk(h):
        mk_base = h * CHUNK

        def body(i, carry):
            base = i * (L * UNROLL)
            for j in range(UNROLL):
                pos = mk_base + base + j * L + lax.iota(jnp.int32, L)
                mi = lax.shift_right_logical(pos, 5)
                cen = plsc.load_gather(cen_v, [mi])
                raw = outa_v[pl.ds(base + j * L, L)]
                outb_v[pl.ds(base + j * L, L)] = raw - cen
            return carry
        lax.fori_loop(0, VECS // UNROLL, body, 0, unroll=False)

    def run_task(ti, carry):
        t = wid + ti * NW

        @pl.when(t < NT)
        def _():
            b = t // CH
            ch = t - b * CH
            is_xyz = ch < 3

            @pl.when(jnp.logical_not(is_xyz))
            def _feat():
                pltpu.sync_copy(feat.at[b, ch - 3], table_v)
                for h in range(NCHUNK):
                    pltpu.sync_copy(idx2.at[b, pl.ds(h * CHUNK, CHUNK)], idx_v)
                    gather_chunk(h)
                    pltpu.sync_copy(outa_v, nf.at[b, ch, pl.ds(h * CHUNK, CHUNK)])

            @pl.when(is_xyz)
            def _xyz():
                pltpu.sync_copy(xyz_t.at[b, ch], table_v)
                pltpu.sync_copy(nxyz_t.at[b, ch], cen_v)
                for h in range(NCHUNK):
                    pltpu.sync_copy(idx2.at[b, pl.ds(h * CHUNK, CHUNK)], idx_v)
                    gather_chunk(h)
                    diff_chunk(h)
                    pltpu.sync_copy(outa_v, gx.at[b, ch, pl.ds(h * CHUNK, CHUNK)])
                    pltpu.sync_copy(outb_v, nf.at[b, ch, pl.ds(h * CHUNK, CHUNK)])

        return carry

    lax.fori_loop(0, TASKS_PER_W, run_task, 0, unroll=False)


@jax.jit
def kernel(xyz, new_xyz, features, idx):
    xyz_t = jnp.transpose(xyz, (0, 2, 1))
    nxyz_t = jnp.transpose(new_xyz, (0, 2, 1))
    idx2 = idx.astype(jnp.int32).reshape(B, MK)

    mesh = plsc.VectorSubcoreMesh(core_axis_name="c", subcore_axis_name="s")
    nf, gx = pl.kernel(
        _qag_kernel,
        out_type=(
            jax.ShapeDtypeStruct((B, CH, MK), jnp.float32),
            jax.ShapeDtypeStruct((B, 3, MK), jnp.float32),
        ),
        mesh=mesh,
        scratch_types=[
            pltpu.VMEM((N,), jnp.float32),
            pltpu.VMEM((CHUNK,), jnp.int32),
            pltpu.VMEM((CHUNK,), jnp.float32),
            pltpu.VMEM((CHUNK,), jnp.float32),
            pltpu.VMEM((M,), jnp.float32),
        ],
        compiler_params=pltpu.CompilerParams(needs_layout_passes=False),
    )(xyz_t, nxyz_t, features, idx2)

    new_features = nf.reshape(B, CH, M, K)
    grouped_xyz = gx.reshape(B, 3, M, K)
    return new_features, grouped_xyz


# speedup vs baseline: 52.0003x; 1.5185x over previous
import functools

import jax
import jax.numpy as jnp
from jax import lax
from jax.experimental import pallas as pl
from jax.experimental.pallas import tpu as pltpu, tpu_sc as plsc

B, N, M, C, K = 8, 65536, 4096, 64, 32
MK = M * K
CH = C + 3
NT = B * CH

NC, NS, L = 2, 16, 16
NW = NC * NS
TASKS_PER_W = (NT + NW - 1) // NW

CHUNK = 8192
NCHUNK = MK // CHUNK
VECS = CHUNK // L
UNROLL = 8


def _qag_kernel(xyz_t, nxyz_t, feat, idx2, nf, gx,
                table_v, idx_v0, idx_v1, outa_v0, outa_v1,
                outb_v0, outb_v1, cen_v,
                sem_t, sem_c, sem_i0, sem_i1, sem_o0, sem_o1, sem_b0, sem_b1):
    cid = lax.axis_index("c")
    sid = lax.axis_index("s")
    wid = sid * NC + cid

    idx_v = (idx_v0, idx_v1)
    outa_v = (outa_v0, outa_v1)
    outb_v = (outb_v0, outb_v1)
    sem_i = (sem_i0, sem_i1)
    sem_o = (sem_o0, sem_o1)
    sem_b = (sem_b0, sem_b1)

    def gather_chunk(iv_ref, oa_ref):
        @plsc.parallel_loop(0, VECS, 1, unroll=UNROLL)
        def _(i):
            iv = iv_ref[pl.ds(i * L, L)]
            oa_ref[pl.ds(i * L, L)] = plsc.load_gather(table_v, [iv])

    def gather_diff_chunk(iv_ref, oa_ref, ob_ref, h):
        mk_base = h * CHUNK

        @plsc.parallel_loop(0, VECS, 1, unroll=UNROLL)
        def _(i):
            iv = iv_ref[pl.ds(i * L, L)]
            raw = plsc.load_gather(table_v, [iv])
            pos = mk_base + i * L + lax.iota(jnp.int32, L)
            mi = lax.shift_right_logical(pos, 5)
            cen = plsc.load_gather(cen_v, [mi])
            oa_ref[pl.ds(i * L, L)] = raw
            ob_ref[pl.ds(i * L, L)] = raw - cen

    def run_task(ti, carry):
        t = wid + ti * NW

        @pl.when(t < NT)
        def _():
            b = t // CH
            ch = t - b * CH
            is_xyz = ch < 3

            def idx_src(h):
                return idx2.at[b, pl.ds(h * CHUNK, CHUNK)]

            @pl.when(jnp.logical_not(is_xyz))
            def _feat():
                tc = pltpu.async_copy(feat.at[b, ch - 3], table_v, sem_t)
                pltpu.async_copy(idx_src(0), idx_v[0], sem_i[0])
                tc.wait()
                for h in range(NCHUNK):
                    p = h % 2
                    if h + 1 < NCHUNK:
                        pltpu.async_copy(idx_src(h + 1), idx_v[(h + 1) % 2],
                                         sem_i[(h + 1) % 2])
                    pltpu.make_async_copy(idx_src(h), idx_v[p], sem_i[p]).wait()
                    if h >= 2:
                        pltpu.make_async_copy(
                            outa_v[p], nf.at[b, ch, pl.ds((h - 2) * CHUNK, CHUNK)],
                            sem_o[p]).wait()
                    gather_chunk(idx_v[p], outa_v[p])
                    pltpu.async_copy(
                        outa_v[p], nf.at[b, ch, pl.ds(h * CHUNK, CHUNK)], sem_o[p])
                for h in (NCHUNK - 2, NCHUNK - 1):
                    p = h % 2
                    pltpu.make_async_copy(
                        outa_v[p], nf.at[b, ch, pl.ds(h * CHUNK, CHUNK)],
                        sem_o[p]).wait()

            @pl.when(is_xyz)
            def _xyz():
                tc = pltpu.async_copy(xyz_t.at[b, ch], table_v, sem_t)
                cc = pltpu.async_copy(nxyz_t.at[b, ch], cen_v, sem_c)
                pltpu.async_copy(idx_src(0), idx_v[0], sem_i[0])
                tc.wait()
                cc.wait()
                for h in range(NCHUNK):
                    p = h % 2
                    if h + 1 < NCHUNK:
                        pltpu.async_copy(idx_src(h + 1), idx_v[(h + 1) % 2],
                                         sem_i[(h + 1) % 2])
                    pltpu.make_async_copy(idx_src(h), idx_v[p], sem_i[p]).wait()
                    if h >= 2:
                        pltpu.make_async_copy(
                            outa_v[p], gx.at[b, ch, pl.ds((h - 2) * CHUNK, CHUNK)],
                            sem_o[p]).wait()
                        pltpu.make_async_copy(
                            outb_v[p], nf.at[b, ch, pl.ds((h - 2) * CHUNK, CHUNK)],
                            sem_b[p]).wait()
                    gather_diff_chunk(idx_v[p], outa_v[p], outb_v[p], h)
                    pltpu.async_copy(
                        outa_v[p], gx.at[b, ch, pl.ds(h * CHUNK, CHUNK)], sem_o[p])
                    pltpu.async_copy(
                        outb_v[p], nf.at[b, ch, pl.ds(h * CHUNK, CHUNK)], sem_b[p])
                for h in (NCHUNK - 2, NCHUNK - 1):
                    p = h % 2
                    pltpu.make_async_copy(
                        outa_v[p], gx.at[b, ch, pl.ds(h * CHUNK, CHUNK)],
                        sem_o[p]).wait()
                    pltpu.make_async_copy(
                        outb_v[p], nf.at[b, ch, pl.ds(h * CHUNK, CHUNK)],
                        sem_b[p]).wait()

        return carry

    lax.fori_loop(0, TASKS_PER_W, run_task, 0, unroll=False)


@jax.jit
def kernel(xyz, new_xyz, features, idx):
    xyz_t = jnp.transpose(xyz, (0, 2, 1))
    nxyz_t = jnp.transpose(new_xyz, (0, 2, 1))
    idx2 = idx.astype(jnp.int32).reshape(B, MK)

    mesh = plsc.VectorSubcoreMesh(core_axis_name="c", subcore_axis_name="s")
    nf, gx = pl.kernel(
        _qag_kernel,
        out_type=(
            jax.ShapeDtypeStruct((B, CH, MK), jnp.float32),
            jax.ShapeDtypeStruct((B, 3, MK), jnp.float32),
        ),
        mesh=mesh,
        scratch_types=[
            pltpu.VMEM((N,), jnp.float32),
            pltpu.VMEM((CHUNK,), jnp.int32),
            pltpu.VMEM((CHUNK,), jnp.int32),
            pltpu.VMEM((CHUNK,), jnp.float32),
            pltpu.VMEM((CHUNK,), jnp.float32),
            pltpu.VMEM((CHUNK,), jnp.float32),
            pltpu.VMEM((CHUNK,), jnp.float32),
            pltpu.VMEM((M,), jnp.float32),
            pltpu.SemaphoreType.DMA,
            pltpu.SemaphoreType.DMA,
            pltpu.SemaphoreType.DMA,
            pltpu.SemaphoreType.DMA,
            pltpu.SemaphoreType.DMA,
            pltpu.SemaphoreType.DMA,
            pltpu.SemaphoreType.DMA,
            pltpu.SemaphoreType.DMA,
        ],
        compiler_params=pltpu.CompilerParams(needs_layout_passes=False),
    )(xyz_t, nxyz_t, features, idx2)

    new_features = nf.reshape(B, CH, M, K)
    grouped_xyz = gx.reshape(B, 3, M, K)
    return new_features, grouped_xyz
